# baseline (speedup 1.0000x reference)
import jax
import jax.numpy as jnp
from jax.experimental import pallas as pl
from jax.experimental.pallas import tpu as pltpu

_SEG = 2500
_NSEG = 4
_F32 = jax.lax.Precision.HIGHEST
_BF = jnp.bfloat16


def _bdot(a, b_ref):
    return jnp.dot(a.astype(_BF), b_ref[...],
                   preferred_element_type=jnp.float32)


def _fwd(x_hbm, wr_ref, br_ref, wc_ref, bc_ref, w1_ref, b1_ref,
         w5_ref, b5_ref, w6_ref, b6_ref, w7_ref, b7_ref, out_ref,
         buf_ref, sem, cum_ref, xacc_ref, phys_ref):
    s = pl.program_id(0)
    nst = pl.num_programs(0)

    def copy(si):
        slot = jax.lax.rem(si, 2)
        return pltpu.make_async_copy(
            x_hbm.at[si // _NSEG, pl.ds(jax.lax.rem(si, _NSEG) * _SEG, _SEG), :],
            buf_ref.at[slot],
            sem.at[slot])

    @pl.when(s == 0)
    def _first():
        copy(0).start()

    @pl.when(s + 1 < nst)
    def _next():
        copy(s + 1).start()

    copy(s).wait()

    g = jax.lax.rem(s, _NSEG)
    x = buf_ref[jax.lax.rem(s, 2)]
    h = jnp.maximum(_bdot(x, wr_ref) + br_ref[...], 0.0)
    ssum = jnp.sum(h, axis=0, keepdims=True)

    @pl.when(g == 0)
    def _reset():
        cum_ref[...] = ssum
        xacc_ref[...] = jnp.zeros_like(xacc_ref)
        phys_ref[...] = x[0:1, 38:53]

    @pl.when(g != 0)
    def _accum():
        cum_ref[...] += ssum

    xacc_ref[...] += jnp.maximum(_bdot(cum_ref[...], wc_ref) + bc_ref[...],
                                 0.0)

    @pl.when(g == _NSEG - 1)
    def _head():
        y = jnp.maximum(_bdot(xacc_ref[...], w1_ref) + b1_ref[...],
                        0.0)
        y = jnp.maximum(_bdot(y, w5_ref) + b5_ref[...], 0.0)
        mv = jnp.dot(y, w6_ref[...], precision=_F32,
                     preferred_element_type=jnp.float32) + b6_ref[...]
        phys = phys_ref[...]
        merged = jnp.concatenate([mv, phys], axis=1)
        fin = jnp.dot(merged, w7_ref[...], precision=_F32,
                      preferred_element_type=jnp.float32) + b7_ref[...]
        out_ref[...] = jnp.concatenate([fin, phys], axis=1)[None]


def kernel(inputs, W_rule, b_rule, W_conv, b_conv, W1, b1, W5, b5, W6, b6,
           W7, b7):
    B, N, F = inputs.shape
    wr = jnp.zeros((F, W_rule.shape[1]), jnp.float32).at[:36, :].set(W_rule)
    row = lambda v: v.reshape(1, -1)

    def full(shape):
        return pl.BlockSpec(shape, lambda s: (0,) * len(shape))

    out = pl.pallas_call(
        _fwd,
        grid=(B * _NSEG,),
        in_specs=[
            pl.BlockSpec(memory_space=pl.ANY),
            full((F, 20)),
            full((1, 20)),
            full(W_conv.shape),
            full((1, 1024)),
            full(W1.shape),
            full((1, 32)),
            full(W5.shape),
            full((1, 16)),
            full(W6.shape),
            full((1, 1)),
            full(W7.shape),
            full((1, 1)),
        ],
        out_specs=pl.BlockSpec((1, 1, 16), lambda s: (s // _NSEG, 0, 0)),
        out_shape=jax.ShapeDtypeStruct((B, 1, 16), jnp.float32),
        scratch_shapes=[
            pltpu.VMEM((2, _SEG, F), jnp.float32),
            pltpu.SemaphoreType.DMA((2,)),
            pltpu.VMEM((1, 20), jnp.float32),
            pltpu.VMEM((1, 1024), jnp.float32),
            pltpu.VMEM((1, 15), jnp.float32),
        ],
    )(inputs, wr.astype(_BF), row(b_rule), W_conv.astype(_BF),
      row(b_conv), W1.astype(_BF), row(b1), W5.astype(_BF), row(b5),
      W6, row(b6), W7, row(b7))
    return out.reshape(B, 16)
